# Initial kernel scaffold; baseline (speedup 1.0000x reference)
import jax
import jax.numpy as jnp
from jax.experimental import pallas as pl


def kernel(x, gate_w, gate_up_w, down_w):
    raise NotImplementedError("write your pallas kernel here")


# speedup vs baseline: 1.0808x; 1.0808x over previous
import functools

import jax
import jax.numpy as jnp
from jax import lax
from jax.experimental import pallas as pl
from jax.experimental.pallas import tpu as pltpu
from jax.experimental.pallas import tpu_sc as plsc

_E = 8
_TOPK = 2
_BM = 128
_FT = 512


def _router(x2d, gate_w, T, D, E):
    def body(x_ref, gw_ref, idx_ref, w_ref):
        x = x_ref[...]
        gw = gw_ref[...]
        logits = lax.dot_general(
            x, gw, (((1,), (1,)), ((), ())), preferred_element_type=jnp.float32
        )
        m = jnp.max(logits, axis=1, keepdims=True)
        p = jnp.exp(logits - m)
        p = p / jnp.sum(p, axis=1, keepdims=True)
        lane = lax.broadcasted_iota(jnp.int32, p.shape, 1)
        m1 = jnp.max(p, axis=1, keepdims=True)
        i1 = jnp.min(jnp.where(p == m1, lane, E), axis=1, keepdims=True)
        pm = jnp.where(lane == i1, -jnp.inf, p)
        m2 = jnp.max(pm, axis=1, keepdims=True)
        i2 = jnp.min(jnp.where(pm == m2, lane, E), axis=1, keepdims=True)
        idx_ref[...] = jnp.where(lane == 0, i1, jnp.where(lane == 1, i2, 0))
        w_ref[...] = jnp.where(lane == 0, m1, jnp.where(lane == 1, m2, 0.0))

    return pl.pallas_call(
        body,
        out_shape=(
            jax.ShapeDtypeStruct((T, E), jnp.int32),
            jax.ShapeDtypeStruct((T, E), jnp.float32),
        ),
    )(x2d, gate_w)


def _sc_gather(x2d, trow, P, D):
    info = plsc.get_sparse_core_info()
    nw = info.num_cores * info.num_subcores
    rw = P // nw
    ch = min(rw, 80)
    nch = rw // ch
    mesh = plsc.VectorSubcoreMesh(core_axis_name="c", subcore_axis_name="s")

    @functools.partial(
        pl.kernel,
        out_type=jax.ShapeDtypeStruct((P, D), jnp.float32),
        mesh=mesh,
        scratch_types=[
            pltpu.VMEM((ch,), jnp.int32),
            pltpu.VMEM((ch, D), jnp.float32),
            pltpu.SemaphoreType.DMA,
        ],
    )
    def k(x_hbm, idx_hbm, out_hbm, idx_v, rows_v, sem):
        wid = lax.axis_index("s") * info.num_cores + lax.axis_index("c")
        base = wid * rw
        for c in range(nch):
            pltpu.sync_copy(idx_hbm.at[pl.ds(base + c * ch, ch)], idx_v)
            pltpu.async_copy(x_hbm.at[idx_v], rows_v, sem).wait()
            pltpu.sync_copy(rows_v, out_hbm.at[pl.ds(base + c * ch, ch)])

    return k(x2d, trow)


def _gemm_pass1(block_eid, xs, gate_up_w, P, D, F, NB, NF):
    def body(eid_ref, x_ref, gwg_ref, gwu_ref, h_ref):
        x = x_ref[...]
        g = jnp.dot(x, gwg_ref[0], preferred_element_type=jnp.float32)
        u = jnp.dot(x, gwu_ref[0], preferred_element_type=jnp.float32)
        h_ref[...] = (g * lax.logistic(g)) * u

    return pl.pallas_call(
        body,
        grid_spec=pltpu.PrefetchScalarGridSpec(
            num_scalar_prefetch=1,
            grid=(NF, NB),
            in_specs=[
                pl.BlockSpec((_BM, D), lambda n, b, eid: (b, 0)),
                pl.BlockSpec((1, D, _FT), lambda n, b, eid: (eid[b], 0, n)),
                pl.BlockSpec((1, D, _FT), lambda n, b, eid: (eid[b], 0, n + NF)),
            ],
            out_specs=pl.BlockSpec((_BM, _FT), lambda n, b, eid: (b, n)),
        ),
        out_shape=jax.ShapeDtypeStruct((P, F), jnp.float32),
    )(block_eid, xs, gate_up_w, gate_up_w)


def _gemm_pass2(block_eid, h, wrow8, down_w, P, D, F, NB):
    def body(eid_ref, h_ref, w_ref, dw_ref, o_ref):
        hw = h_ref[...] * w_ref[...][:, 0:1]
        o_ref[...] = jnp.dot(hw, dw_ref[0], preferred_element_type=jnp.float32)

    return pl.pallas_call(
        body,
        grid_spec=pltpu.PrefetchScalarGridSpec(
            num_scalar_prefetch=1,
            grid=(NB,),
            in_specs=[
                pl.BlockSpec((_BM, F), lambda b, eid: (b, 0)),
                pl.BlockSpec((_BM, 8), lambda b, eid: (b, 0)),
                pl.BlockSpec((1, F, D), lambda b, eid: (eid[b], 0, 0)),
            ],
            out_specs=pl.BlockSpec((_BM, D), lambda b, eid: (b, 0)),
        ),
        out_shape=jax.ShapeDtypeStruct((P, D), jnp.float32),
    )(block_eid, h, wrow8, down_w)


def _sc_combine(os_rows, pos1, pos2, T, D):
    info = plsc.get_sparse_core_info()
    nw = info.num_cores * info.num_subcores
    tpw = T // nw
    ct = min(tpw, 32)
    nch = tpw // ct
    nvec = ct * D // 16
    mesh = plsc.VectorSubcoreMesh(core_axis_name="c", subcore_axis_name="s")

    @functools.partial(
        pl.kernel,
        out_type=jax.ShapeDtypeStruct((T, D), jnp.float32),
        mesh=mesh,
        scratch_types=[
            pltpu.VMEM((ct,), jnp.int32),
            pltpu.VMEM((ct,), jnp.int32),
            pltpu.VMEM((ct, D), jnp.float32),
            pltpu.VMEM((ct, D), jnp.float32),
            pltpu.SemaphoreType.DMA,
            pltpu.SemaphoreType.DMA,
        ],
    )
    def k(os_hbm, p1_hbm, p2_hbm, out_hbm, p1_v, p2_v, r1, r2, sem1, sem2):
        wid = lax.axis_index("s") * info.num_cores + lax.axis_index("c")
        base = wid * tpw
        for c in range(nch):
            tb = base + c * ct
            pltpu.sync_copy(p1_hbm.at[pl.ds(tb, ct)], p1_v)
            pltpu.sync_copy(p2_hbm.at[pl.ds(tb, ct)], p2_v)
            cp1 = pltpu.async_copy(os_hbm.at[p1_v], r1, sem1)
            cp2 = pltpu.async_copy(os_hbm.at[p2_v], r2, sem2)
            cp1.wait()
            cp2.wait()

            def add_body(j, _):
                t = j // (D // 16)
                f = (j % (D // 16)) * 16
                sl = pl.ds(f, 16)
                r1[t, sl] = r1[t, sl] + r2[t, sl]
                return 0

            lax.fori_loop(0, nvec, add_body, 0)
            pltpu.sync_copy(r1, out_hbm.at[pl.ds(tb, ct)])

    return k(os_rows, pos1, pos2)


def kernel(x, gate_w, gate_up_w, down_w):
    B, S, D = x.shape
    E, _, F2 = gate_up_w.shape
    F = F2 // 2
    T = B * S
    x2d = x.reshape(T, D)

    idx8, w8 = _router(x2d, gate_w, T, D, E)
    e1 = idx8[:, 0]
    e2 = idx8[:, 1]
    w1 = w8[:, 0]
    w2 = w8[:, 1]

    a = jnp.concatenate([e1, e2])
    onehot = (a[:, None] == jnp.arange(E, dtype=jnp.int32)[None, :]).astype(jnp.int32)
    csum = jnp.cumsum(onehot, axis=0)
    counts = csum[-1]
    rank = jnp.take_along_axis(csum, a[:, None], axis=1)[:, 0] - 1
    pc = ((counts + _BM - 1) // _BM) * _BM
    ends = jnp.cumsum(pc)
    off = ends - pc
    pos = off[a] + rank

    nw = 32
    NB = (2 * T) // _BM + E - 1
    while (NB * _BM) % (8 * nw) != 0:
        NB += 1
    P = NB * _BM
    NF = F // _FT

    tokid = jnp.arange(2 * T, dtype=jnp.int32) % T
    trow = jnp.zeros((P,), jnp.int32).at[pos].set(tokid)
    wflat = jnp.concatenate([w1, w2])
    wrow = jnp.zeros((P,), jnp.float32).at[pos].set(wflat)
    wrow8 = jnp.broadcast_to(wrow[:, None], (P, 8))
    block_eid = jnp.minimum(
        jnp.searchsorted(ends, jnp.arange(NB, dtype=jnp.int32) * _BM, side="right"),
        E - 1,
    ).astype(jnp.int32)
    pos1 = pos[:T].astype(jnp.int32)
    pos2 = pos[T:].astype(jnp.int32)

    xs = _sc_gather(x2d, trow, P, D)

    h = _gemm_pass1(block_eid, xs, gate_up_w, P, D, F, NB, NF)
    os_rows = _gemm_pass2(block_eid, h, wrow8, down_w, P, D, F, NB)

    out = _sc_combine(os_rows, pos1, pos2, T, D)
    return out.reshape(B, S, D)
